# baseline (speedup 1.0000x reference)
import functools
import jax
import jax.numpy as jnp
from jax import lax
from jax.experimental import pallas as pl
from jax.experimental.pallas import tpu as pltpu
from jax.experimental.pallas import tpu_sc as plsc

D_M = 1024
CHUNK = 64


def _make_sc(batch, seq_len, d_model):
    n_workers = 32
    rows = batch * seq_len
    rows_per_w = rows // n_workers
    n_chunks = rows_per_w // CHUNK
    mesh = plsc.VectorSubcoreMesh(
        core_axis_name="c", subcore_axis_name="s", num_cores=2, num_subcores=16
    )

    @functools.partial(
        pl.kernel,
        out_type=jax.ShapeDtypeStruct((rows, d_model), jnp.float32),
        mesh=mesh,
        scratch_types=[
            pltpu.VMEM((CHUNK, d_model), jnp.float32),
            pltpu.VMEM((CHUNK,), jnp.int32),
            pltpu.SemaphoreType.DMA,
        ],
    )
    def k(x_hbm, pos_hbm, ids_hbm, out_hbm, xbuf, idxbuf, sem):
        wid = lax.axis_index("s") * 2 + lax.axis_index("c")
        row0 = wid * rows_per_w
        seq0 = row0 % seq_len

        def body(c, _):
            r = row0 + c * CHUNK
            s = seq0 + c * CHUNK
            pltpu.sync_copy(x_hbm.at[pl.ds(r, CHUNK)], xbuf)
            pltpu.sync_copy(ids_hbm.at[pl.ds(s, CHUNK)], idxbuf)
            pltpu.async_copy(pos_hbm.at[idxbuf], xbuf, sem, add=True).wait()
            pltpu.sync_copy(xbuf, out_hbm.at[pl.ds(r, CHUNK)])
            return ()

        lax.fori_loop(0, n_chunks, body, ())

    return k


def kernel(x, pos_table):
    batch, seq_len, d_model = x.shape
    x2 = x.reshape(batch * seq_len, d_model)
    ids = jnp.arange(seq_len, dtype=jnp.int32)
    out = _make_sc(batch, seq_len, d_model)(x2, pos_table, ids)
    return out.reshape(batch, seq_len, d_model)
